# baseline (speedup 1.0000x reference)
import functools

import jax
import jax.numpy as jnp
from jax import lax
from jax.experimental import pallas as pl
from jax.experimental.pallas import tpu as pltpu
from jax.experimental.pallas import tpu_sc as plsc

VOCAB = 1000000
EMBED = 64
HIDDEN = 64
B = 4096
T = 50
G3 = 3 * HIDDEN

BB = 512
NC = 2
NS = 16
NW = NC * NS
CH = 800


def _gather_sc(emb, idx_flat):
    TB = idx_flat.shape[0]
    b_per_w = TB // NW
    n_ch = b_per_w // CH
    mesh = plsc.VectorSubcoreMesh(core_axis_name="c", subcore_axis_name="s")

    @functools.partial(
        pl.kernel,
        out_type=jax.ShapeDtypeStruct((TB, EMBED), jnp.float32),
        mesh=mesh,
        scratch_types=[
            pltpu.VMEM((CH,), jnp.int32),
            pltpu.VMEM((CH, EMBED), jnp.float32),
            pltpu.SemaphoreType.DMA,
        ],
        compiler_params=pltpu.CompilerParams(use_tc_tiling_on_sc=False),
    )
    def gather_kernel(table_hbm, i_hbm, o_hbm, idx_v, rows_v, sem):
        wid = lax.axis_index("s") * NC + lax.axis_index("c")

        @pl.loop(0, n_ch)
        def _(c):
            base = wid * b_per_w + c * CH
            pltpu.sync_copy(i_hbm.at[pl.ds(base, CH)], idx_v)
            pltpu.async_copy(table_hbm.at[idx_v], rows_v, sem).wait()
            pltpu.sync_copy(rows_v, o_hbm.at[pl.ds(base, CH)])

    return gather_kernel(emb, idx_flat)


def _gru_body(xs_ref, wih_ref, whh_ref, bih_ref, bhh_ref, fcw_ref, fcb_ref,
              out_ref, gi_ref):
    x2 = xs_ref[...].reshape(T * BB, EMBED)
    gi_ref[...] = (
        jnp.dot(x2, wih_ref[...], preferred_element_type=jnp.float32)
        + bih_ref[...]
    ).reshape(T, BB, G3)

    whh = whh_ref[...]
    bhh = bhh_ref[...]

    def step(t, h):
        gi_t = gi_ref[t]
        gh = jnp.dot(h, whh, preferred_element_type=jnp.float32) + bhh
        r = jax.nn.sigmoid(gi_t[:, 0:HIDDEN] + gh[:, 0:HIDDEN])
        z = jax.nn.sigmoid(gi_t[:, HIDDEN:2 * HIDDEN] + gh[:, HIDDEN:2 * HIDDEN])
        n = jnp.tanh(gi_t[:, 2 * HIDDEN:] + r * gh[:, 2 * HIDDEN:])
        return (1.0 - z) * n + z * h

    h = lax.fori_loop(0, T, step, jnp.zeros((BB, HIDDEN), jnp.float32))
    out_ref[...] = (
        jnp.dot(h, fcw_ref[...], preferred_element_type=jnp.float32)
        + fcb_ref[...]
    )


def _gru_tc(xs, wih_t, whh_t, bih, bhh, fcw_t, fcb):
    return pl.pallas_call(
        _gru_body,
        grid=(B // BB,),
        in_specs=[
            pl.BlockSpec((T, BB, EMBED), lambda i: (0, i, 0)),
            pl.BlockSpec((EMBED, G3), lambda i: (0, 0)),
            pl.BlockSpec((HIDDEN, G3), lambda i: (0, 0)),
            pl.BlockSpec((1, G3), lambda i: (0, 0)),
            pl.BlockSpec((1, G3), lambda i: (0, 0)),
            pl.BlockSpec((HIDDEN, HIDDEN), lambda i: (0, 0)),
            pl.BlockSpec((1, HIDDEN), lambda i: (0, 0)),
        ],
        out_specs=pl.BlockSpec((BB, HIDDEN), lambda i: (i, 0)),
        out_shape=jax.ShapeDtypeStruct((B, HIDDEN), jnp.float32),
        scratch_shapes=[pltpu.VMEM((T, BB, G3), jnp.float32)],
    )(xs, wih_t, whh_t, bih, bhh, fcw_t, fcb)


def kernel(x, emb, w_ih, w_hh, b_ih, b_hh, fc_w, fc_b):
    idx = x.astype(jnp.int32).T.reshape(-1)
    rows = _gather_sc(emb, idx)
    xs = rows.reshape(T, B, EMBED)
    return _gru_tc(
        xs,
        w_ih.T,
        w_hh.T,
        b_ih.reshape(1, G3),
        b_hh.reshape(1, G3),
        fc_w.T,
        fc_b.reshape(1, HIDDEN),
    )


# speedup vs baseline: 1.2309x; 1.2309x over previous
import functools

import jax
import jax.numpy as jnp
from jax import lax
from jax.experimental import pallas as pl
from jax.experimental.pallas import tpu as pltpu
from jax.experimental.pallas import tpu_sc as plsc

VOCAB = 1000000
EMBED = 64
HIDDEN = 64
B = 4096
T = 50
G3 = 3 * HIDDEN
B2 = B // 2
PW = 2 * EMBED
PG = 2 * G3

NC = 2
NS = 16
NW = NC * NS
CH2 = 400


def _gather_sc(emb, idx_even, idx_odd):
    TB2 = idx_even.shape[0]
    b_per_w = TB2 // NW
    n_ch = b_per_w // CH2
    mesh = plsc.VectorSubcoreMesh(core_axis_name="c", subcore_axis_name="s")

    @functools.partial(
        pl.kernel,
        out_type=jax.ShapeDtypeStruct((TB2, PW), jnp.float32),
        mesh=mesh,
        scratch_types=[
            pltpu.VMEM((CH2,), jnp.int32),
            pltpu.VMEM((CH2,), jnp.int32),
            pltpu.VMEM((CH2, EMBED), jnp.float32),
            pltpu.VMEM((CH2, EMBED), jnp.float32),
            pltpu.SemaphoreType.DMA,
        ],
        compiler_params=pltpu.CompilerParams(use_tc_tiling_on_sc=False),
    )
    def gather_kernel(table_hbm, ie_hbm, io_hbm, o_hbm,
                      idx_e_v, idx_o_v, rows_e, rows_o, sem):
        wid = lax.axis_index("s") * NC + lax.axis_index("c")

        @pl.loop(0, n_ch)
        def _(c):
            base = wid * b_per_w + c * CH2
            pltpu.sync_copy(ie_hbm.at[pl.ds(base, CH2)], idx_e_v)
            pltpu.sync_copy(io_hbm.at[pl.ds(base, CH2)], idx_o_v)
            ge = pltpu.async_copy(table_hbm.at[idx_e_v], rows_e, sem)
            go = pltpu.async_copy(table_hbm.at[idx_o_v], rows_o, sem)
            ge.wait()
            go.wait()
            pltpu.sync_copy(
                rows_e, o_hbm.at[pl.ds(base, CH2), pl.ds(0, EMBED)])
            pltpu.sync_copy(
                rows_o, o_hbm.at[pl.ds(base, CH2), pl.ds(EMBED, EMBED)])

    return gather_kernel(emb, idx_even, idx_odd)


def _gru_body(xs_hbm, wih_ref, whh_ref, bih_ref, bhh_ref, fcw_ref, fcb_ref,
              out_ref, x0, x1, h_ref, sem0, sem1):
    pltpu.make_async_copy(xs_hbm.at[0], x0, sem0).start()
    pltpu.make_async_copy(xs_hbm.at[1], x1, sem1).start()
    h_ref[...] = jnp.zeros((B2, PW), jnp.float32)
    wih = wih_ref[...]
    whh = whh_ref[...]
    bih = bih_ref[...]
    bhh = bhh_ref[...]

    def gru_step(xt, h):
        gi = jnp.dot(xt, wih, preferred_element_type=jnp.float32) + bih
        gh = jnp.dot(h, whh, preferred_element_type=jnp.float32) + bhh
        r = jax.nn.sigmoid(gi[:, 0:128] + gh[:, 0:128])
        z = jax.nn.sigmoid(gi[:, 128:256] + gh[:, 128:256])
        n = jnp.tanh(gi[:, 256:384] + r * gh[:, 256:384])
        return (1.0 - z) * n + z * h

    def pair(i, carry):
        t0 = 2 * i
        pltpu.make_async_copy(xs_hbm.at[t0], x0, sem0).wait()
        h_ref[...] = gru_step(x0[...], h_ref[...])

        @pl.when(i < (T // 2) - 1)
        def _():
            pltpu.make_async_copy(xs_hbm.at[t0 + 2], x0, sem0).start()

        pltpu.make_async_copy(xs_hbm.at[t0 + 1], x1, sem1).wait()
        h_ref[...] = gru_step(x1[...], h_ref[...])

        @pl.when(i < (T // 2) - 1)
        def _():
            pltpu.make_async_copy(xs_hbm.at[t0 + 3], x1, sem1).start()

        return carry

    lax.fori_loop(0, T // 2, pair, 0)
    out_ref[...] = (
        jnp.dot(h_ref[...], fcw_ref[...], preferred_element_type=jnp.float32)
        + fcb_ref[...]
    )


def _gru_tc(xs2, wih2, whh2, bih2, bhh2, fcw2, fcb2):
    return pl.pallas_call(
        _gru_body,
        in_specs=[
            pl.BlockSpec(memory_space=pl.ANY),
            pl.BlockSpec(memory_space=pltpu.MemorySpace.VMEM),
            pl.BlockSpec(memory_space=pltpu.MemorySpace.VMEM),
            pl.BlockSpec(memory_space=pltpu.MemorySpace.VMEM),
            pl.BlockSpec(memory_space=pltpu.MemorySpace.VMEM),
            pl.BlockSpec(memory_space=pltpu.MemorySpace.VMEM),
            pl.BlockSpec(memory_space=pltpu.MemorySpace.VMEM),
        ],
        out_specs=pl.BlockSpec(memory_space=pltpu.MemorySpace.VMEM),
        out_shape=jax.ShapeDtypeStruct((B2, PW), jnp.float32),
        scratch_shapes=[
            pltpu.VMEM((B2, PW), jnp.float32),
            pltpu.VMEM((B2, PW), jnp.float32),
            pltpu.VMEM((B2, PW), jnp.float32),
            pltpu.SemaphoreType.DMA,
            pltpu.SemaphoreType.DMA,
        ],
    )(xs2, wih2, whh2, bih2, bhh2, fcw2, fcb2)


def _pair_weights(wT):
    z = jnp.zeros((EMBED, HIDDEN), wT.dtype)
    blocks = []
    for g in range(3):
        wg = wT[:, g * HIDDEN:(g + 1) * HIDDEN]
        top = jnp.concatenate([wg, z], axis=1)
        bot = jnp.concatenate([z, wg], axis=1)
        blocks.append(jnp.concatenate([top, bot], axis=0))
    return jnp.concatenate(blocks, axis=1)


def _pair_bias(b):
    return jnp.tile(b.reshape(3, 1, HIDDEN), (1, 2, 1)).reshape(1, PG)


def kernel(x, emb, w_ih, w_hh, b_ih, b_hh, fc_w, fc_b):
    idx = x.astype(jnp.int32).T.reshape(-1, 2)
    rows2 = _gather_sc(emb, idx[:, 0], idx[:, 1])
    xs2 = rows2.reshape(T, B2, PW)

    fcwT = fc_w.T
    z = jnp.zeros((HIDDEN, HIDDEN), fcwT.dtype)
    fcw2 = jnp.concatenate(
        [jnp.concatenate([fcwT, z], axis=1),
         jnp.concatenate([z, fcwT], axis=1)], axis=0)

    out2 = _gru_tc(
        xs2,
        _pair_weights(w_ih.T),
        _pair_weights(w_hh.T),
        _pair_bias(b_ih),
        _pair_bias(b_hh),
        fcw2,
        jnp.tile(fc_b.reshape(1, HIDDEN), (1, 2)),
    )
    return out2.reshape(B, HIDDEN)
